# baseline (speedup 1.0000x reference)
import functools

import jax
import jax.numpy as jnp
from jax import lax
from jax.experimental import pallas as pl
from jax.experimental.pallas import tpu as pltpu
from jax.experimental.pallas import tpu_sc as plsc

NUM_FIELDS = 26
VOCAB = 100000
EMBED = 16
B = 16384
MLP_IN = NUM_FIELDS * EMBED
H1, H2 = 128, 64

BF = B * NUM_FIELDS
NV = NUM_FIELDS * VOCAB
NW = 32
PER_W = BF // NW
CROWS = 1664
NCH = PER_W // CROWS
NG = CROWS // 16

DET_BLK = 81920
DET_C = -(-NV // DET_BLK)


def _detile_body(in_ref, l_ref, *out_refs):
    for d in range(EMBED):
        out_refs[d][...] = in_ref[d, :]
    out_refs[EMBED][...] = l_ref[0, :]


def _detile(tet, tl1):
    return pl.pallas_call(
        _detile_body,
        grid=(DET_C,),
        in_specs=[
            pl.BlockSpec((EMBED, DET_BLK), lambda c: (0, c)),
            pl.BlockSpec((1, DET_BLK), lambda c: (0, c)),
        ],
        out_specs=[
            pl.BlockSpec((DET_BLK,), lambda c: (c,))
            for _ in range(EMBED + 1)
        ],
        out_shape=[
            jax.ShapeDtypeStruct((NV,), jnp.float32)
            for _ in range(EMBED + 1)
        ],
    )(tet, tl1)


def _sc_gather(flat_idx, planes, tlf):
    mesh = plsc.VectorSubcoreMesh(core_axis_name="c", subcore_axis_name="s")

    @functools.partial(
        pl.kernel,
        mesh=mesh,
        compiler_params=pltpu.CompilerParams(
            use_tc_tiling_on_sc=False, needs_layout_passes=False),
        out_type=[
            jax.ShapeDtypeStruct((NW * NCH, CROWS * EMBED), jnp.float32),
            jax.ShapeDtypeStruct((NW * NCH, CROWS), jnp.float32),
        ],
        scratch_types=[
            pltpu.VMEM((NCH, CROWS), jnp.int32),
            pltpu.VMEM((2, EMBED, CROWS), jnp.float32),
            pltpu.VMEM((2, CROWS * EMBED), jnp.float32),
            pltpu.VMEM((2, CROWS), jnp.float32),
            pltpu.SemaphoreType.DMA,
            pltpu.SemaphoreType.DMA,
            pltpu.SemaphoreType.DMA,
            pltpu.SemaphoreType.DMA,
        ],
    )
    def k(idx_hbm, *rest):
        te_hbms = rest[:EMBED]
        (tl_hbm, oute_hbm, outl_hbm, idx_v, pbuf, obuf, lbuf,
         sem_e0, sem_e1, sem_l0, sem_l1) = rest[EMBED:]
        wid = lax.axis_index("s") * 2 + lax.axis_index("c")
        base = wid * NCH
        pltpu.sync_copy(idx_hbm.at[wid], idx_v)
        sem_e = (sem_e0, sem_e1)
        sem_l = (sem_l0, sem_l1)
        lane16 = lax.iota(jnp.int32, 16) * EMBED

        def fire(j, s):
            cl = pltpu.async_copy(tl_hbm.at[idx_v.at[j]], lbuf.at[s],
                                  sem_l[s])
            ces = [
                pltpu.async_copy(te_hbms[d].at[idx_v.at[j]], pbuf.at[s, d],
                                 sem_e[s])
                for d in range(EMBED)
            ]
            return ces, cl

        def transpose_chunk(s):
            def body(g, _):
                kbase = g * (16 * EMBED)
                for d in range(EMBED):
                    vals = pbuf[s, d, pl.ds(g * 16, 16)]
                    plsc.store_scatter(obuf.at[s], [kbase + lane16 + d], vals)
                return 0
            lax.fori_loop(0, NG, body, 0)

        prev = None
        for j in range(NCH):
            s = j % 2
            cur = (j, *fire(j, s))
            if prev is not None:
                pj, pces, pcl = prev
                ps = pj % 2
                for c in pces:
                    c.wait()
                pcl.wait()
                transpose_chunk(ps)
                pltpu.sync_copy(obuf.at[ps], oute_hbm.at[base + pj])
                pltpu.sync_copy(lbuf.at[ps], outl_hbm.at[base + pj])
            prev = cur
        pj, pces, pcl = prev
        ps = pj % 2
        for c in pces:
            c.wait()
        pcl.wait()
        transpose_chunk(ps)
        pltpu.sync_copy(obuf.at[ps], oute_hbm.at[base + pj])
        pltpu.sync_copy(lbuf.at[ps], outl_hbm.at[base + pj])

    return k(flat_idx, *planes, tlf)


def _tc_body(e_ref, lin_ref, w1_ref, b1_ref, w2_ref, b2_ref, w3_ref, b3_ref,
             s_ref, out_ref):
    e = e_ref[...]
    h = jnp.maximum(
        jnp.dot(e, w1_ref[...], preferred_element_type=jnp.float32)
        + b1_ref[...], 0.0)
    h = jnp.maximum(
        jnp.dot(h, w2_ref[...], preferred_element_type=jnp.float32)
        + b2_ref[...], 0.0)
    dnn = jnp.sum(h * w3_ref[...], axis=1, keepdims=True) + b3_ref[...]
    s = s_ref[...]
    se = jnp.dot(e, s, preferred_element_type=jnp.float32)
    ss = jnp.dot(e * e, s, preferred_element_type=jnp.float32)
    fm = 0.5 * jnp.sum(se * se - ss, axis=1, keepdims=True)
    linear = jnp.sum(lin_ref[...], axis=1, keepdims=True)
    out_ref[...] = jax.nn.sigmoid(dnn + fm + linear)


def _tc_forward(e, lin, W1, b1, W2, b2, w3r, b3):
    BT = 2048
    grid = B // BT
    s_mat = jnp.tile(jnp.eye(EMBED, dtype=jnp.float32), (NUM_FIELDS, 1))
    return pl.pallas_call(
        _tc_body,
        grid=(grid,),
        in_specs=[
            pl.BlockSpec((BT, MLP_IN), lambda i: (i, 0)),
            pl.BlockSpec((BT, NUM_FIELDS), lambda i: (i, 0)),
            pl.BlockSpec((MLP_IN, H1), lambda i: (0, 0)),
            pl.BlockSpec((1, H1), lambda i: (0, 0)),
            pl.BlockSpec((H1, H2), lambda i: (0, 0)),
            pl.BlockSpec((1, H2), lambda i: (0, 0)),
            pl.BlockSpec((1, H2), lambda i: (0, 0)),
            pl.BlockSpec((1, 1), lambda i: (0, 0)),
            pl.BlockSpec((MLP_IN, EMBED), lambda i: (0, 0)),
        ],
        out_specs=pl.BlockSpec((BT, 1), lambda i: (i, 0)),
        out_shape=jax.ShapeDtypeStruct((B, 1), jnp.float32),
    )(e, lin, W1, b1, W2, b2, w3r, b3, s_mat)


def kernel(x, table_e, table_l, W1, b1, W2, b2, W3, b3):
    offsets = jnp.arange(NUM_FIELDS, dtype=jnp.int32) * VOCAB
    flat_idx = (x.astype(jnp.int32) + offsets[None, :]).reshape(NW, NCH, CROWS)
    outs = _detile(table_e.T, table_l.T)
    planes, tlf = outs[:EMBED], outs[EMBED]
    embed, lin = _sc_gather(flat_idx, planes, tlf)
    out = _tc_forward(
        embed.reshape(B, MLP_IN),
        lin.reshape(B, NUM_FIELDS),
        W1, b1.reshape(1, H1), W2, b2.reshape(1, H2), W3.reshape(1, H2),
        b3.reshape(1, 1),
    )
    return out


# speedup vs baseline: 1.0193x; 1.0193x over previous
import functools

import jax
import jax.numpy as jnp
from jax import lax
from jax.experimental import pallas as pl
from jax.experimental.pallas import tpu as pltpu
from jax.experimental.pallas import tpu_sc as plsc

NUM_FIELDS = 26
VOCAB = 100000
EMBED = 16
B = 16384
MLP_IN = NUM_FIELDS * EMBED
H1, H2 = 128, 64

BF = B * NUM_FIELDS
NV = NUM_FIELDS * VOCAB
NW = 32
PER_W = BF // NW
CROWS = 832
NCH = PER_W // CROWS
NG = CROWS // 16

DET_BLK = 81920
DET_C = -(-NV // DET_BLK)


def _detile_body(in_ref, l_ref, *out_refs):
    for d in range(EMBED):
        out_refs[d][...] = in_ref[d, :]
    out_refs[EMBED][...] = l_ref[0, :]


def _detile(tet, tl1):
    return pl.pallas_call(
        _detile_body,
        grid=(DET_C,),
        in_specs=[
            pl.BlockSpec((EMBED, DET_BLK), lambda c: (0, c)),
            pl.BlockSpec((1, DET_BLK), lambda c: (0, c)),
        ],
        out_specs=[
            pl.BlockSpec((DET_BLK,), lambda c: (c,))
            for _ in range(EMBED + 1)
        ],
        out_shape=[
            jax.ShapeDtypeStruct((NV,), jnp.float32)
            for _ in range(EMBED + 1)
        ],
    )(tet, tl1)


def _sc_gather(flat_idx, planes, tlf):
    mesh = plsc.VectorSubcoreMesh(core_axis_name="c", subcore_axis_name="s")

    @functools.partial(
        pl.kernel,
        mesh=mesh,
        compiler_params=pltpu.CompilerParams(
            use_tc_tiling_on_sc=False, needs_layout_passes=False),
        out_type=[
            jax.ShapeDtypeStruct((NW * NCH, CROWS * EMBED), jnp.float32),
            jax.ShapeDtypeStruct((NW * NCH, CROWS), jnp.float32),
        ],
        scratch_types=[
            pltpu.VMEM((NCH, CROWS), jnp.int32),
            pltpu.VMEM((3, EMBED, CROWS), jnp.float32),
            pltpu.VMEM((3, CROWS * EMBED), jnp.float32),
            pltpu.VMEM((3, CROWS), jnp.float32),
            pltpu.SemaphoreType.DMA,
            pltpu.SemaphoreType.DMA,
            pltpu.SemaphoreType.DMA,
            pltpu.SemaphoreType.DMA,
            pltpu.SemaphoreType.DMA,
            pltpu.SemaphoreType.DMA,
        ],
    )
    def k(idx_hbm, *rest):
        te_hbms = rest[:EMBED]
        (tl_hbm, oute_hbm, outl_hbm, idx_v, pbuf, obuf, lbuf,
         sem_e0, sem_e1, sem_e2, sem_l0, sem_l1, sem_l2) = rest[EMBED:]
        wid = lax.axis_index("s") * 2 + lax.axis_index("c")
        base = wid * NCH
        pltpu.sync_copy(idx_hbm.at[wid], idx_v)
        sem_e = (sem_e0, sem_e1, sem_e2)
        sem_l = (sem_l0, sem_l1, sem_l2)
        lane16 = lax.iota(jnp.int32, 16) * EMBED

        def fire(j, s):
            cl = pltpu.async_copy(tl_hbm.at[idx_v.at[j]], lbuf.at[s],
                                  sem_l[s])
            ces = [
                pltpu.async_copy(te_hbms[d].at[idx_v.at[j]], pbuf.at[s, d],
                                 sem_e[s])
                for d in range(EMBED)
            ]
            return ces, cl

        def transpose_chunk(s):
            def body(g, _):
                kbase = g * (16 * EMBED)
                for d in range(EMBED):
                    vals = pbuf[s, d, pl.ds(g * 16, 16)]
                    plsc.store_scatter(obuf.at[s], [kbase + lane16 + d], vals)
                return 0
            lax.fori_loop(0, NG, body, 0)

        def drain(t, cps):
            pces, pcl = cps[t]
            ps = t % 3
            for c in pces:
                c.wait()
            pcl.wait()
            transpose_chunk(ps)
            pltpu.sync_copy(obuf.at[ps], oute_hbm.at[base + t])
            pltpu.sync_copy(lbuf.at[ps], outl_hbm.at[base + t])

        cps = {}
        for j in range(NCH):
            cps[j] = fire(j, j % 3)
            if j >= 2:
                drain(j - 2, cps)
        drain(NCH - 2, cps)
        drain(NCH - 1, cps)

    return k(flat_idx, *planes, tlf)


def _tc_body(e_ref, lin_ref, w1_ref, b1_ref, w2_ref, b2_ref, w3_ref, b3_ref,
             s_ref, out_ref):
    e = e_ref[...]
    h = jnp.maximum(
        jnp.dot(e, w1_ref[...], preferred_element_type=jnp.float32)
        + b1_ref[...], 0.0)
    h = jnp.maximum(
        jnp.dot(h, w2_ref[...], preferred_element_type=jnp.float32)
        + b2_ref[...], 0.0)
    dnn = jnp.sum(h * w3_ref[...], axis=1, keepdims=True) + b3_ref[...]
    s = s_ref[...]
    se = jnp.dot(e, s, preferred_element_type=jnp.float32)
    ss = jnp.dot(e * e, s, preferred_element_type=jnp.float32)
    fm = 0.5 * jnp.sum(se * se - ss, axis=1, keepdims=True)
    linear = jnp.sum(lin_ref[...], axis=1, keepdims=True)
    out_ref[...] = jax.nn.sigmoid(dnn + fm + linear)


def _tc_forward(e, lin, W1, b1, W2, b2, w3r, b3):
    BT = 2048
    grid = B // BT
    s_mat = jnp.tile(jnp.eye(EMBED, dtype=jnp.float32), (NUM_FIELDS, 1))
    return pl.pallas_call(
        _tc_body,
        grid=(grid,),
        in_specs=[
            pl.BlockSpec((BT, MLP_IN), lambda i: (i, 0)),
            pl.BlockSpec((BT, NUM_FIELDS), lambda i: (i, 0)),
            pl.BlockSpec((MLP_IN, H1), lambda i: (0, 0)),
            pl.BlockSpec((1, H1), lambda i: (0, 0)),
            pl.BlockSpec((H1, H2), lambda i: (0, 0)),
            pl.BlockSpec((1, H2), lambda i: (0, 0)),
            pl.BlockSpec((1, H2), lambda i: (0, 0)),
            pl.BlockSpec((1, 1), lambda i: (0, 0)),
            pl.BlockSpec((MLP_IN, EMBED), lambda i: (0, 0)),
        ],
        out_specs=pl.BlockSpec((BT, 1), lambda i: (i, 0)),
        out_shape=jax.ShapeDtypeStruct((B, 1), jnp.float32),
    )(e, lin, W1, b1, W2, b2, w3r, b3, s_mat)


def kernel(x, table_e, table_l, W1, b1, W2, b2, W3, b3):
    offsets = jnp.arange(NUM_FIELDS, dtype=jnp.int32) * VOCAB
    flat_idx = (x.astype(jnp.int32) + offsets[None, :]).reshape(NW, NCH, CROWS)
    outs = _detile(table_e.T, table_l.T)
    planes, tlf = outs[:EMBED], outs[EMBED]
    embed, lin = _sc_gather(flat_idx, planes, tlf)
    out = _tc_forward(
        embed.reshape(B, MLP_IN),
        lin.reshape(B, NUM_FIELDS),
        W1, b1.reshape(1, H1), W2, b2.reshape(1, H2), W3.reshape(1, H2),
        b3.reshape(1, 1),
    )
    return out


# speedup vs baseline: 1.0197x; 1.0004x over previous
import functools

import jax
import jax.numpy as jnp
from jax import lax
from jax.experimental import pallas as pl
from jax.experimental.pallas import tpu as pltpu
from jax.experimental.pallas import tpu_sc as plsc

NUM_FIELDS = 26
VOCAB = 100000
EMBED = 16
B = 16384
MLP_IN = NUM_FIELDS * EMBED
H1, H2 = 128, 64

BF = B * NUM_FIELDS
NV = NUM_FIELDS * VOCAB
NW = 32
PER_W = BF // NW
CROWS = 416
NCH = PER_W // CROWS
NG = CROWS // 16

DET_BLK = 81920
DET_C = -(-NV // DET_BLK)


def _detile_body(in_ref, l_ref, *out_refs):
    for d in range(EMBED):
        out_refs[d][...] = in_ref[d, :]
    out_refs[EMBED][...] = l_ref[0, :]


def _detile(tet, tl1):
    return pl.pallas_call(
        _detile_body,
        grid=(DET_C,),
        in_specs=[
            pl.BlockSpec((EMBED, DET_BLK), lambda c: (0, c)),
            pl.BlockSpec((1, DET_BLK), lambda c: (0, c)),
        ],
        out_specs=[
            pl.BlockSpec((DET_BLK,), lambda c: (c,))
            for _ in range(EMBED + 1)
        ],
        out_shape=[
            jax.ShapeDtypeStruct((NV,), jnp.float32)
            for _ in range(EMBED + 1)
        ],
    )(tet, tl1)


def _sc_gather(flat_idx, planes, tlf):
    mesh = plsc.VectorSubcoreMesh(core_axis_name="c", subcore_axis_name="s")

    @functools.partial(
        pl.kernel,
        mesh=mesh,
        compiler_params=pltpu.CompilerParams(
            use_tc_tiling_on_sc=False, needs_layout_passes=False),
        out_type=[
            jax.ShapeDtypeStruct((NW * NCH, CROWS * EMBED), jnp.float32),
            jax.ShapeDtypeStruct((NW * NCH, CROWS), jnp.float32),
        ],
        scratch_types=[
            pltpu.VMEM((NCH, CROWS), jnp.int32),
            pltpu.VMEM((3, EMBED, CROWS), jnp.float32),
            pltpu.VMEM((3, CROWS * EMBED), jnp.float32),
            pltpu.VMEM((3, CROWS), jnp.float32),
            pltpu.SemaphoreType.DMA,
            pltpu.SemaphoreType.DMA,
            pltpu.SemaphoreType.DMA,
            pltpu.SemaphoreType.DMA,
            pltpu.SemaphoreType.DMA,
            pltpu.SemaphoreType.DMA,
        ],
    )
    def k(idx_hbm, *rest):
        te_hbms = rest[:EMBED]
        (tl_hbm, oute_hbm, outl_hbm, idx_v, pbuf, obuf, lbuf,
         sem_e0, sem_e1, sem_e2, sem_l0, sem_l1, sem_l2) = rest[EMBED:]
        wid = lax.axis_index("s") * 2 + lax.axis_index("c")
        base = wid * NCH
        pltpu.sync_copy(idx_hbm.at[wid], idx_v)
        sem_e = (sem_e0, sem_e1, sem_e2)
        sem_l = (sem_l0, sem_l1, sem_l2)
        lane16 = lax.iota(jnp.int32, 16) * EMBED

        def fire(j, s):
            cl = pltpu.async_copy(tl_hbm.at[idx_v.at[j]], lbuf.at[s],
                                  sem_l[s])
            ces = [
                pltpu.async_copy(te_hbms[d].at[idx_v.at[j]], pbuf.at[s, d],
                                 sem_e[s])
                for d in range(EMBED)
            ]
            return ces, cl

        def transpose_chunk(s):
            def body(g, _):
                kbase = g * (16 * EMBED)
                for d in range(EMBED):
                    vals = pbuf[s, d, pl.ds(g * 16, 16)]
                    plsc.store_scatter(obuf.at[s], [kbase + lane16 + d], vals)
                return 0
            lax.fori_loop(0, NG, body, 0)

        def drain(t, cps):
            pces, pcl = cps[t]
            ps = t % 3
            for c in pces:
                c.wait()
            pcl.wait()
            transpose_chunk(ps)
            pltpu.sync_copy(obuf.at[ps], oute_hbm.at[base + t])
            pltpu.sync_copy(lbuf.at[ps], outl_hbm.at[base + t])

        cps = {}
        for j in range(NCH):
            cps[j] = fire(j, j % 3)
            if j >= 2:
                drain(j - 2, cps)
        drain(NCH - 2, cps)
        drain(NCH - 1, cps)

    return k(flat_idx, *planes, tlf)


def _tc_body(e_ref, lin_ref, w1_ref, b1_ref, w2_ref, b2_ref, w3_ref, b3_ref,
             s_ref, out_ref):
    e = e_ref[...]
    h = jnp.maximum(
        jnp.dot(e, w1_ref[...], preferred_element_type=jnp.float32)
        + b1_ref[...], 0.0)
    h = jnp.maximum(
        jnp.dot(h, w2_ref[...], preferred_element_type=jnp.float32)
        + b2_ref[...], 0.0)
    dnn = jnp.sum(h * w3_ref[...], axis=1, keepdims=True) + b3_ref[...]
    s = s_ref[...]
    se = jnp.dot(e, s, preferred_element_type=jnp.float32)
    ss = jnp.dot(e * e, s, preferred_element_type=jnp.float32)
    fm = 0.5 * jnp.sum(se * se - ss, axis=1, keepdims=True)
    linear = jnp.sum(lin_ref[...], axis=1, keepdims=True)
    out_ref[...] = jax.nn.sigmoid(dnn + fm + linear)


def _tc_forward(e, lin, W1, b1, W2, b2, w3r, b3):
    BT = 2048
    grid = B // BT
    s_mat = jnp.tile(jnp.eye(EMBED, dtype=jnp.float32), (NUM_FIELDS, 1))
    return pl.pallas_call(
        _tc_body,
        grid=(grid,),
        in_specs=[
            pl.BlockSpec((BT, MLP_IN), lambda i: (i, 0)),
            pl.BlockSpec((BT, NUM_FIELDS), lambda i: (i, 0)),
            pl.BlockSpec((MLP_IN, H1), lambda i: (0, 0)),
            pl.BlockSpec((1, H1), lambda i: (0, 0)),
            pl.BlockSpec((H1, H2), lambda i: (0, 0)),
            pl.BlockSpec((1, H2), lambda i: (0, 0)),
            pl.BlockSpec((1, H2), lambda i: (0, 0)),
            pl.BlockSpec((1, 1), lambda i: (0, 0)),
            pl.BlockSpec((MLP_IN, EMBED), lambda i: (0, 0)),
        ],
        out_specs=pl.BlockSpec((BT, 1), lambda i: (i, 0)),
        out_shape=jax.ShapeDtypeStruct((B, 1), jnp.float32),
    )(e, lin, W1, b1, W2, b2, w3r, b3, s_mat)


def kernel(x, table_e, table_l, W1, b1, W2, b2, W3, b3):
    offsets = jnp.arange(NUM_FIELDS, dtype=jnp.int32) * VOCAB
    flat_idx = (x.astype(jnp.int32) + offsets[None, :]).reshape(NW, NCH, CROWS)
    outs = _detile(table_e.T, table_l.T)
    planes, tlf = outs[:EMBED], outs[EMBED]
    embed, lin = _sc_gather(flat_idx, planes, tlf)
    out = _tc_forward(
        embed.reshape(B, MLP_IN),
        lin.reshape(B, NUM_FIELDS),
        W1, b1.reshape(1, H1), W2, b2.reshape(1, H2), W3.reshape(1, H2),
        b3.reshape(1, 1),
    )
    return out
